# baseline (speedup 1.0000x reference)
import functools

import jax
import jax.numpy as jnp
from jax import lax
from jax.experimental import pallas as pl
from jax.experimental.pallas import tpu as pltpu
from jax.experimental.pallas import tpu_sc as plsc

NUM_ACTIVATIONS = 65536
D_MODEL = 1024
K = 512

NC = 2
NS = 16
L = 16
NW = NC * NS
KPW = K // NW
CHUNKS = D_MODEL // L


def _sc_partials(top_indices, weight, W_dec):
  mesh = plsc.VectorSubcoreMesh(core_axis_name="c", subcore_axis_name="s")

  @functools.partial(
      pl.kernel,
      out_type=jax.ShapeDtypeStruct((NW, D_MODEL), jnp.float32),
      mesh=mesh,
      scratch_types=[
          pltpu.VMEM((KPW,), jnp.int32),
          pltpu.VMEM((KPW,), jnp.float32),
          pltpu.VMEM((KPW, D_MODEL), jnp.float32),
          pltpu.VMEM((D_MODEL,), jnp.float32),
          pltpu.SemaphoreType.DMA,
          pltpu.SemaphoreType.DMA,
      ],
  )
  def body(idx_hbm, w_hbm, wdec_hbm, out_hbm, idx_v, wv, rows_v, acc_v,
           sem_w, sem_rows):
    wid = lax.axis_index("s") * NC + lax.axis_index("c")
    base = wid * KPW
    pltpu.sync_copy(idx_hbm.at[pl.ds(base, KPW)], idx_v)
    cp_w = pltpu.async_copy(w_hbm.at[idx_v], wv, sem_w)
    cp_r = pltpu.async_copy(wdec_hbm.at[idx_v], rows_v, sem_rows)
    cp_w.wait()
    cp_r.wait()

    w_reg = jnp.maximum(wv[...], 0.0)
    dn = lax.GatherDimensionNumbers(
        offset_dims=(), collapsed_slice_dims=(0,), start_index_map=(0,))
    bws = []
    for i in range(KPW):
      lane = jnp.full((L, 1), i, dtype=jnp.int32)
      bws.append(lax.gather(w_reg, lane, dn, (1,),
                            mode=lax.GatherScatterMode.PROMISE_IN_BOUNDS))

    def chunk_body(c, carry):
      sl = pl.ds(c * L, L)
      acc = bws[0] * rows_v[0, sl]
      for i in range(1, KPW):
        acc = acc + bws[i] * rows_v[i, sl]
      acc_v[sl] = acc
      return carry

    lax.fori_loop(0, CHUNKS, chunk_body, 0)
    pltpu.sync_copy(acc_v, out_hbm.at[wid])

  return body(top_indices, weight, W_dec)


def _tc_combine(partials, b_dec_2d):
  def body(p_ref, b_ref, o_ref):
    o_ref[...] = jnp.sum(p_ref[...], axis=0, keepdims=True) + b_ref[...]

  return pl.pallas_call(
      body,
      out_shape=jax.ShapeDtypeStruct((1, D_MODEL), jnp.float32),
  )(partials, b_dec_2d)


def kernel(top_indices, weight, W_dec, b_dec):
  idx = top_indices.astype(jnp.int32)
  partials = _sc_partials(idx, weight, W_dec)
  return _tc_combine(partials, b_dec.reshape(1, D_MODEL))


# speedup vs baseline: 1.0162x; 1.0162x over previous
import functools

import jax
import jax.numpy as jnp
from jax import lax
from jax.experimental import pallas as pl
from jax.experimental.pallas import tpu as pltpu
from jax.experimental.pallas import tpu_sc as plsc

NUM_ACTIVATIONS = 65536
D_MODEL = 1024
K = 512

NC = 2
NS = 16
L = 16
DH = D_MODEL // NC
KPS = K // NS
CHUNKS = DH // L


def _sc_func_vec(top_indices, weight, W_dec, b_dec):
  mesh = plsc.VectorSubcoreMesh(core_axis_name="c", subcore_axis_name="s")

  @functools.partial(
      pl.kernel,
      out_type=jax.ShapeDtypeStruct((1, D_MODEL), jnp.float32),
      mesh=mesh,
      scratch_types=[
          pltpu.VMEM((KPS,), jnp.int32),
          pltpu.VMEM((KPS,), jnp.float32),
          pltpu.VMEM((KPS, DH), jnp.float32),
          pltpu.VMEM((1, DH), jnp.float32),
          pltpu.VMEM((1,), jnp.int32),
          pltpu.VMEM_SHARED((1, DH), jnp.float32),
          pltpu.SemaphoreType.DMA,
          pltpu.SemaphoreType.DMA,
      ],
  )
  def body(idx_hbm, w_hbm, wdec_hbm, bdec_hbm, zero_hbm, out_hbm, idx_v, wv,
           rows_v, acc_v, zero_v, shared_acc, sem_w, sem_rows):
    cid = lax.axis_index("c")
    sid = lax.axis_index("s")
    dbase = cid * DH

    pltpu.sync_copy(idx_hbm.at[pl.ds(sid * KPS, KPS)], idx_v)
    cp_w = pltpu.async_copy(w_hbm.at[idx_v], wv, sem_w)
    cp_r = pltpu.async_copy(wdec_hbm.at[idx_v, pl.ds(dbase, DH)], rows_v,
                            sem_rows)
    pltpu.sync_copy(zero_hbm, zero_v)

    @pl.when(sid == 0)
    def _():
      pltpu.sync_copy(bdec_hbm.at[pl.ds(0, 1), pl.ds(dbase, DH)], acc_v)
      pltpu.sync_copy(acc_v, shared_acc)

    cp_w.wait()
    dn = lax.GatherDimensionNumbers(
        offset_dims=(), collapsed_slice_dims=(0,), start_index_map=(0,))
    bws = []
    for h in range(KPS // L):
      w_reg = jnp.maximum(wv[pl.ds(h * L, L)], 0.0)
      for i in range(L):
        lane = jnp.full((L, 1), i, dtype=jnp.int32)
        bws.append(lax.gather(w_reg, lane, dn, (1,),
                              mode=lax.GatherScatterMode.PROMISE_IN_BOUNDS))
    cp_r.wait()

    def chunk_body(c, carry):
      sl = pl.ds(c * L, L)
      acc = bws[0] * rows_v[0, sl]
      for i in range(1, KPS):
        acc = acc + bws[i] * rows_v[i, sl]
      acc_v[0, sl] = acc
      return carry

    lax.fori_loop(0, CHUNKS, chunk_body, 0)

    plsc.subcore_barrier()
    pltpu.sync_copy(acc_v, shared_acc.at[zero_v], add=True)
    plsc.subcore_barrier()

    @pl.when(sid == 0)
    def _():
      pltpu.sync_copy(shared_acc, acc_v)
      pltpu.sync_copy(acc_v, out_hbm.at[pl.ds(0, 1), pl.ds(dbase, DH)])

  zero = jnp.zeros((1,), jnp.int32)
  return body(top_indices, weight, W_dec, b_dec.reshape(1, D_MODEL), zero)


def kernel(top_indices, weight, W_dec, b_dec):
  idx = top_indices.astype(jnp.int32)
  return _sc_func_vec(idx, weight, W_dec, b_dec)


# speedup vs baseline: 1.0256x; 1.0092x over previous
import functools

import jax
import jax.numpy as jnp
from jax import lax
from jax.experimental import pallas as pl
from jax.experimental.pallas import tpu as pltpu
from jax.experimental.pallas import tpu_sc as plsc

NUM_ACTIVATIONS = 65536
D_MODEL = 1024
K = 512

NC = 2
NS = 16
L = 16
DH = D_MODEL // NC
KPS = K // NS
G = KPS // L
CHUNKS = DH // L

_DN = lax.GatherDimensionNumbers(
    offset_dims=(), collapsed_slice_dims=(0,), start_index_map=(0,))


def _broadcasts(w_reg):
  out = []
  for i in range(L):
    lane = jnp.full((L, 1), i, dtype=jnp.int32)
    out.append(lax.gather(w_reg, lane, _DN, (1,),
                          mode=lax.GatherScatterMode.PROMISE_IN_BOUNDS))
  return out


def _tree_sum(terms):
  while len(terms) > 1:
    nxt = [a + b for a, b in zip(terms[::2], terms[1::2])]
    if len(terms) % 2:
      nxt.append(terms[-1])
    terms = nxt
  return terms[0]


def _sc_func_vec(top_indices, weight, W_dec, b_dec, zero):
  mesh = plsc.VectorSubcoreMesh(core_axis_name="c", subcore_axis_name="s")

  @functools.partial(
      pl.kernel,
      out_type=jax.ShapeDtypeStruct((1, D_MODEL), jnp.float32),
      mesh=mesh,
      scratch_types=[
          pltpu.VMEM((KPS,), jnp.int32),
          pltpu.VMEM((KPS,), jnp.float32),
          pltpu.VMEM((L, DH), jnp.float32),
          pltpu.VMEM((L, DH), jnp.float32),
          pltpu.VMEM((1, DH), jnp.float32),
          pltpu.VMEM((1,), jnp.int32),
          pltpu.VMEM_SHARED((1, DH), jnp.float32),
          pltpu.SemaphoreType.DMA,
          pltpu.SemaphoreType.DMA,
          pltpu.SemaphoreType.DMA,
      ],
  )
  def body(idx_hbm, w_hbm, wdec_hbm, bdec_hbm, zero_hbm, out_hbm, idx_v, wv,
           rows0_v, rows1_v, acc_v, zero_v, shared_acc, sem_w, sem_r0,
           sem_r1):
    cid = lax.axis_index("c")
    sid = lax.axis_index("s")
    dbase = cid * DH

    pltpu.sync_copy(idx_hbm.at[pl.ds(sid * KPS, KPS)], idx_v)
    cp_w = pltpu.async_copy(w_hbm.at[idx_v], wv, sem_w)
    cp_r0 = pltpu.async_copy(
        wdec_hbm.at[idx_v.at[pl.ds(0, L)], pl.ds(dbase, DH)], rows0_v, sem_r0)
    cp_r1 = pltpu.async_copy(
        wdec_hbm.at[idx_v.at[pl.ds(L, L)], pl.ds(dbase, DH)], rows1_v, sem_r1)
    pltpu.sync_copy(zero_hbm, zero_v)

    @pl.when(sid == 0)
    def _():
      pltpu.sync_copy(bdec_hbm.at[pl.ds(0, 1), pl.ds(dbase, DH)], acc_v)
      pltpu.sync_copy(acc_v, shared_acc)

    cp_w.wait()
    bws0 = _broadcasts(jnp.maximum(wv[pl.ds(0, L)], 0.0))
    bws1 = _broadcasts(jnp.maximum(wv[pl.ds(L, L)], 0.0))

    cp_r0.wait()

    def group0_body(c, carry):
      sl = pl.ds(c * L, L)
      acc_v[0, sl] = _tree_sum([bws0[i] * rows0_v[i, sl] for i in range(L)])
      return carry

    lax.fori_loop(0, CHUNKS, group0_body, 0)

    cp_r1.wait()

    def group1_body(c, carry):
      sl = pl.ds(c * L, L)
      acc_v[0, sl] = acc_v[0, sl] + _tree_sum(
          [bws1[i] * rows1_v[i, sl] for i in range(L)])
      return carry

    lax.fori_loop(0, CHUNKS, group1_body, 0)

    plsc.subcore_barrier()
    pltpu.sync_copy(acc_v, shared_acc.at[zero_v], add=True)
    plsc.subcore_barrier()

    @pl.when(sid == 0)
    def _():
      pltpu.sync_copy(shared_acc, acc_v)
      pltpu.sync_copy(acc_v, out_hbm.at[pl.ds(0, 1), pl.ds(dbase, DH)])

  return body(top_indices, weight, W_dec, b_dec, zero)


def kernel(top_indices, weight, W_dec, b_dec):
  idx = top_indices.astype(jnp.int32)
  zero = jnp.zeros((1,), jnp.int32)
  return _sc_func_vec(idx, weight, W_dec, b_dec.reshape(1, D_MODEL), zero)
